# baseline (speedup 1.0000x reference)
import functools

import jax
import jax.numpy as jnp
from jax import lax
from jax.experimental import pallas as pl
from jax.experimental.pallas import tpu as pltpu
from jax.experimental.pallas import tpu_sc as plsc

MAXP = 64
K = 9
D = 256
TAU = 0.07
NC, NS, L = 2, 16, 16
NW = NC * NS
GROUP = 64
NB = GROUP // L


def _rsqrt16(x):
    i = plsc.bitcast(x, jnp.int32)
    y = plsc.bitcast(jnp.int32(0x5F3759DF) - (i >> 1), jnp.float32)
    for _ in range(3):
        y = y * (1.5 - 0.5 * x * y * y)
    return y


def _make_sc_call(T):
    TPW = T // NW
    NG = TPW // GROUP
    mesh = plsc.VectorSubcoreMesh(
        core_axis_name="c", subcore_axis_name="s",
        num_cores=NC, num_subcores=NS)

    @functools.partial(
        pl.kernel,
        out_type=jax.ShapeDtypeStruct((T * K,), jnp.float32),
        mesh=mesh,
        scratch_types=[
            pltpu.VMEM((GROUP * D,), jnp.float32),
            pltpu.VMEM((GROUP * D,), jnp.float32),
            pltpu.VMEM((L * D,), jnp.float32),
            pltpu.VMEM((D * L,), jnp.float32),
            pltpu.VMEM((GROUP * K,), jnp.float32),
            pltpu.SemaphoreType.DMA,
            pltpu.SemaphoreType.DMA,
        ],
        compiler_params=pltpu.CompilerParams(needs_layout_passes=False),
    )
    def sc_call(z_hbm, p_hbm, out_hbm, zb0, zb1, pbuf, ptbuf, obuf, sem0, sem1):
        wid = lax.axis_index("s") * NC + lax.axis_index("c")
        lane = lax.iota(jnp.int32, L)
        lane_d = lane * D
        lane_k = lane * K

        pltpu.sync_copy(p_hbm.at[pl.ds(0, L * D)], pbuf)

        def _nsq(d, acc):
            v = plsc.load_gather(pbuf, [lane_d + d])
            return acc + v * v

        nsq = lax.fori_loop(0, D, _nsq, jnp.zeros((L,), jnp.float32))
        pinv = _rsqrt16(jnp.maximum(nsq, 1e-24)) * (1.0 / TAU)

        def _transpose(d, c):
            v = plsc.load_gather(pbuf, [lane_d + d])
            plsc.store_scatter(ptbuf, [lane + d * L], v * pinv)
            return c

        lax.fori_loop(0, D, _transpose, 0)

        zbufs = (zb0, zb1)
        sems = (sem0, sem1)
        zbase = wid * TPW * D
        obase = wid * TPW * K

        def _start(g, slot):
            return pltpu.async_copy(
                z_hbm.at[pl.ds(zbase + g * GROUP * D, GROUP * D)],
                zbufs[slot], sems[slot])

        cps = [_start(0, 0), None]
        for g in range(NG):
            cur = g & 1
            if g + 1 < NG:
                cps[1 - cur] = _start(g + 1, 1 - cur)
            cps[cur].wait()
            zb = zbufs[cur]

            def _dot(d, accs, zb=zb):
                pvec = ptbuf[pl.ds(d * L, L)]
                pv = [pvec[k] for k in range(K)]
                new = []
                for b in range(NB):
                    zv = plsc.load_gather(zb, [lane_d + (b * L * D + d)])
                    a = accs[b]
                    new.append(tuple(
                        [a[k] + zv * pv[k] for k in range(K)]
                        + [a[K] + zv * zv]))
                return tuple(new)

            zero = jnp.zeros((L,), jnp.float32)
            init = tuple(tuple(zero for _ in range(K + 1)) for _ in range(NB))
            accs = lax.fori_loop(0, D, _dot, init)

            for b in range(NB):
                a = accs[b]
                zinv = _rsqrt16(jnp.maximum(a[K], 1e-24))
                for k in range(K):
                    plsc.store_scatter(
                        obuf, [lane_k + (b * L * K + k)], a[k] * zinv)
            pltpu.sync_copy(
                obuf, out_hbm.at[pl.ds(obase + g * GROUP * K, GROUP * K)])

    return sc_call


def kernel(hidden_z, prototypes):
    B, S, _ = hidden_z.shape
    T = B * S
    out = _make_sc_call(T)(hidden_z.reshape(T * D), prototypes.reshape(MAXP * D))
    return out.reshape(B, S, K)


# speedup vs baseline: 3.5951x; 3.5951x over previous
import functools

import jax
import jax.numpy as jnp
from jax import lax
from jax.experimental import pallas as pl
from jax.experimental.pallas import tpu as pltpu
from jax.experimental.pallas import tpu_sc as plsc

MAXP = 64
K = 9
D = 256
TAU = 0.07
NC, NS, L = 2, 16, 16
NW = NC * NS
GROUP = 64
NB = GROUP // L
T_SC = 2048
BT = 2048


def _rsqrt16(x):
    i = plsc.bitcast(x, jnp.int32)
    y = plsc.bitcast(jnp.int32(0x5F3759DF) - (i >> 1), jnp.float32)
    for _ in range(3):
        y = y * (1.5 - 0.5 * x * y * y)
    return y


def _make_sc_call(total_t, t_sc):
    off = total_t - t_sc
    TPW = t_sc // NW
    NG = TPW // GROUP
    mesh = plsc.VectorSubcoreMesh(
        core_axis_name="c", subcore_axis_name="s",
        num_cores=NC, num_subcores=NS)

    @functools.partial(
        pl.kernel,
        out_type=jax.ShapeDtypeStruct((t_sc * K,), jnp.float32),
        mesh=mesh,
        scratch_types=[
            pltpu.VMEM((GROUP * D,), jnp.float32),
            pltpu.VMEM((GROUP * D,), jnp.float32),
            pltpu.VMEM((L * D,), jnp.float32),
            pltpu.VMEM((D * K * L,), jnp.float32),
            pltpu.VMEM((GROUP * K,), jnp.float32),
            pltpu.SemaphoreType.DMA,
            pltpu.SemaphoreType.DMA,
        ],
        compiler_params=pltpu.CompilerParams(needs_layout_passes=False),
    )
    def sc_call(z_hbm, p_hbm, out_hbm, zb0, zb1, pbuf, ptbuf, obuf, sem0, sem1):
        wid = lax.axis_index("s") * NC + lax.axis_index("c")
        lane = lax.iota(jnp.int32, L)
        lane_d = lane * D
        lane_k = lane * K

        pltpu.sync_copy(p_hbm.at[pl.ds(0, L * D)], pbuf)

        def _nsq(d, acc):
            v = plsc.load_gather(pbuf, [lane_d + d])
            return acc + v * v

        nsq = lax.fori_loop(0, D, _nsq, jnp.zeros((L,), jnp.float32))
        pinv = _rsqrt16(jnp.maximum(nsq, 1e-24)) * (1.0 / TAU)

        def _expand(d, c):
            v = plsc.load_gather(pbuf, [lane_d + d]) * pinv
            for k in range(K):
                ptbuf[pl.ds(d * (K * L) + k * L, L)] = jnp.broadcast_to(
                    v[k], (L,))
            return c

        lax.fori_loop(0, D, _expand, 0)

        zbufs = (zb0, zb1)
        sems = (sem0, sem1)
        zbase = (off + wid * TPW) * D
        obase = wid * TPW * K

        def _start(g, slot):
            return pltpu.async_copy(
                z_hbm.at[pl.ds(zbase + g * GROUP * D, GROUP * D)],
                zbufs[slot], sems[slot])

        cps = [_start(0, 0), None]
        for g in range(NG):
            cur = g & 1
            if g + 1 < NG:
                cps[1 - cur] = _start(g + 1, 1 - cur)
            cps[cur].wait()
            zb = zbufs[cur]

            def _dot(d, accs, zb=zb):
                zvs = [plsc.load_gather(zb, [lane_d + (b * L * D + d)])
                       for b in range(NB)]
                new = [list(a) for a in accs]
                for k in range(K):
                    pv = ptbuf[pl.ds(d * (K * L) + k * L, L)]
                    for b in range(NB):
                        new[b][k] = accs[b][k] + zvs[b] * pv
                for b in range(NB):
                    new[b][K] = accs[b][K] + zvs[b] * zvs[b]
                return tuple(tuple(n) for n in new)

            zero = jnp.zeros((L,), jnp.float32)
            init = tuple(tuple(zero for _ in range(K + 1)) for _ in range(NB))
            accs = lax.fori_loop(0, D, _dot, init)

            for b in range(NB):
                a = accs[b]
                zinv = _rsqrt16(jnp.maximum(a[K], 1e-24))
                for k in range(K):
                    plsc.store_scatter(
                        obuf, [lane_k + (b * L * K + k)], a[k] * zinv)
            pltpu.sync_copy(
                obuf, out_hbm.at[pl.ds(obase + g * GROUP * K, GROUP * K)])

    return sc_call


def _tc_body(p_ref, z_ref, o_ref):
    p = p_ref[:K, :]
    pn = p * lax.rsqrt(
        jnp.maximum(jnp.sum(p * p, axis=-1, keepdims=True), 1e-24))
    z = z_ref[...]
    zinv = lax.rsqrt(
        jnp.maximum(jnp.sum(z * z, axis=-1, keepdims=True), 1e-24))
    o_ref[...] = (
        jnp.dot(z, pn.T, preferred_element_type=jnp.float32)
        * zinv * (1.0 / TAU))


def _tc_call(z2d, prototypes, t_tc):
    return pl.pallas_call(
        _tc_body,
        grid=(t_tc // BT,),
        in_specs=[
            pl.BlockSpec((MAXP, D), lambda i: (0, 0)),
            pl.BlockSpec((BT, D), lambda i: (i, 0)),
        ],
        out_specs=pl.BlockSpec((BT, K), lambda i: (i, 0)),
        out_shape=jax.ShapeDtypeStruct((t_tc, K), jnp.float32),
    )(prototypes, z2d)


def kernel(hidden_z, prototypes):
    B, S, _ = hidden_z.shape
    T = B * S
    t_tc = T - T_SC
    zf = hidden_z.reshape(T * D)
    pf = prototypes.reshape(MAXP * D)
    out_sc = _make_sc_call(T, T_SC)(zf, pf)
    out_tc = _tc_call(hidden_z.reshape(T, D), prototypes, t_tc)
    out = jnp.concatenate([out_tc, out_sc.reshape(T_SC, K)], axis=0)
    return out.reshape(B, S, K)


# speedup vs baseline: 5.4637x; 1.5198x over previous
import functools

import jax
import jax.numpy as jnp
from jax import lax
from jax.experimental import pallas as pl
from jax.experimental.pallas import tpu as pltpu
from jax.experimental.pallas import tpu_sc as plsc

MAXP = 64
K = 9
D = 256
TAU = 0.07
NC, NS, L = 2, 16, 16
NW = NC * NS
GROUP = 64
NB = GROUP // L
T_SC = 2048
BT = 2048


def _rsqrt16(x):
    i = plsc.bitcast(x, jnp.int32)
    y = plsc.bitcast(jnp.int32(0x5F3759DF) - (i >> 1), jnp.float32)
    for _ in range(3):
        y = y * (1.5 - 0.5 * x * y * y)
    return y


def _make_sc_call(total_t, t_sc):
    off = total_t - t_sc
    TPW = t_sc // NW
    NG = TPW // GROUP
    mesh = plsc.VectorSubcoreMesh(
        core_axis_name="c", subcore_axis_name="s",
        num_cores=NC, num_subcores=NS)

    @functools.partial(
        pl.kernel,
        out_type=jax.ShapeDtypeStruct((t_sc * K,), jnp.float32),
        mesh=mesh,
        scratch_types=[
            pltpu.VMEM((GROUP, D), jnp.float32),
            pltpu.VMEM((GROUP, D), jnp.float32),
            pltpu.VMEM((L, D), jnp.float32),
            pltpu.VMEM((D * K * L,), jnp.float32),
            pltpu.VMEM((GROUP * K,), jnp.float32),
            pltpu.SemaphoreType.DMA,
            pltpu.SemaphoreType.DMA,
        ],
        compiler_params=pltpu.CompilerParams(needs_layout_passes=False),
    )
    def sc_call(z_hbm, p_hbm, out_hbm, zb0, zb1, pbuf, ptbuf, obuf, sem0, sem1):
        wid = lax.axis_index("s") * NC + lax.axis_index("c")
        lane = lax.iota(jnp.int32, L)
        lane_k = lane * K
        lane_l = lane * L

        pltpu.sync_copy(p_hbm.at[pl.ds(0, L)], pbuf)

        def _nsq(d, acc):
            v = plsc.load_gather(pbuf, [lane, jnp.full((L,), d, jnp.int32)])
            return acc + v * v

        nsq = lax.fori_loop(0, D, _nsq, jnp.zeros((L,), jnp.float32))
        pinv = _rsqrt16(jnp.maximum(nsq, 1e-24)) * (1.0 / TAU)

        def _expand(d, c):
            v = plsc.load_gather(
                pbuf, [lane, jnp.full((L,), d, jnp.int32)]) * pinv
            base = lane_l + d * (K * L)
            for l in range(L):
                plsc.store_scatter(ptbuf, [base + l], v)
            return c

        lax.fori_loop(0, D, _expand, 0)

        zbufs = (zb0, zb1)
        sems = (sem0, sem1)
        tbase = off + wid * TPW
        obase = wid * TPW * K

        def _start(g, slot):
            return pltpu.async_copy(
                z_hbm.at[pl.ds(tbase + g * GROUP, GROUP)],
                zbufs[slot], sems[slot])

        cps = [_start(0, 0), None]
        for g in range(NG):
            cur = g & 1
            if g + 1 < NG:
                cps[1 - cur] = _start(g + 1, 1 - cur)
            cps[cur].wait()
            zb = zbufs[cur]

            for h in range(2):
                rows = [lane + (2 * h + b) * L for b in range(2)]

                def _dot(d, accs, zb=zb, rows=rows):
                    dsp = jnp.full((L,), d, jnp.int32)
                    zvs = [plsc.load_gather(zb, [r, dsp]) for r in rows]
                    new = [list(a) for a in accs]
                    for k in range(K):
                        pv = ptbuf[pl.ds(d * (K * L) + k * L, L)]
                        for b in range(2):
                            new[b][k] = accs[b][k] + zvs[b] * pv
                    for b in range(2):
                        new[b][K] = accs[b][K] + zvs[b] * zvs[b]
                    return tuple(tuple(n) for n in new)

                zero = jnp.zeros((L,), jnp.float32)
                init = tuple(
                    tuple(zero for _ in range(K + 1)) for _ in range(2))
                accs = lax.fori_loop(0, D, _dot, init)

                for b in range(2):
                    a = accs[b]
                    zinv = _rsqrt16(jnp.maximum(a[K], 1e-24))
                    for k in range(K):
                        plsc.store_scatter(
                            obuf,
                            [lane_k + ((2 * h + b) * L * K + k)],
                            a[k] * zinv)
            pltpu.sync_copy(
                obuf, out_hbm.at[pl.ds(obase + g * GROUP * K, GROUP * K)])

    return sc_call


def _tc_body(p_ref, z_ref, o_ref):
    p = p_ref[:K, :]
    pn = p * lax.rsqrt(
        jnp.maximum(jnp.sum(p * p, axis=-1, keepdims=True), 1e-24))
    z = z_ref[...]
    zinv = lax.rsqrt(
        jnp.maximum(jnp.sum(z * z, axis=-1, keepdims=True), 1e-24))
    o_ref[...] = (
        jnp.dot(z, pn.T, preferred_element_type=jnp.float32)
        * zinv * (1.0 / TAU))


def _tc_call(z2d, prototypes, t_tc):
    return pl.pallas_call(
        _tc_body,
        grid=(t_tc // BT,),
        in_specs=[
            pl.BlockSpec((MAXP, D), lambda i: (0, 0)),
            pl.BlockSpec((BT, D), lambda i: (i, 0)),
        ],
        out_specs=pl.BlockSpec((BT, K), lambda i: (i, 0)),
        out_shape=jax.ShapeDtypeStruct((t_tc, K), jnp.float32),
    )(prototypes, z2d)


def kernel(hidden_z, prototypes):
    B, S, _ = hidden_z.shape
    T = B * S
    t_tc = T - T_SC
    z2d = hidden_z.reshape(T, D)
    out_sc = _make_sc_call(T, T_SC)(z2d, prototypes)
    out_tc = _tc_call(z2d, prototypes, t_tc)
    out = jnp.concatenate([out_tc, out_sc.reshape(T_SC, K)], axis=0)
    return out.reshape(B, S, K)
